# baseline (speedup 1.0000x reference)
import functools

import jax
import jax.numpy as jnp
from jax import lax
from jax.experimental import pallas as pl
from jax.experimental.pallas import tpu as pltpu
from jax.experimental.pallas import tpu_sc as plsc

L = 16
P = 48


def _make_sc_resample(HW_in, HW_out, C, H, W):
    info = plsc.get_sparse_core_info()
    NW = info.num_cores * info.num_subcores
    per_w = HW_out // NW
    n_chunks = per_w // P

    mesh = plsc.VectorSubcoreMesh(core_axis_name="c", subcore_axis_name="s")

    @functools.partial(
        pl.kernel,
        mesh=mesh,
        out_type=jax.ShapeDtypeStruct((HW_out, C), jnp.float32),
        scratch_types=[
            pltpu.VMEM((P,), jnp.float32),
            pltpu.VMEM((P,), jnp.float32),
            pltpu.VMEM((4, P), jnp.int32),
            pltpu.VMEM((4, P), jnp.float32),
            pltpu.VMEM((4, P, C), jnp.float32),
            pltpu.VMEM((P, C), jnp.float32),
            pltpu.SemaphoreType.DMA,
        ],
    )
    def body(xt_hbm, xs_hbm, ys_hbm, out_hbm, xsv, ysv, idxv, wv, rows, outv, sem):
        cid = lax.axis_index("c")
        sid = lax.axis_index("s")
        wid = sid * info.num_cores + cid
        base_w = wid * per_w

        def chunk_body(ci, carry):
            base = base_w + ci * P
            pltpu.sync_copy(xs_hbm.at[pl.ds(base, P)], xsv)
            pltpu.sync_copy(ys_hbm.at[pl.ds(base, P)], ysv)
            for i in range(P // L):
                sl = pl.ds(i * L, L)
                xs16 = xsv[sl]
                ys16 = ysv[sl]
                x0 = xs16.astype(jnp.int32)
                y0 = ys16.astype(jnp.int32)
                dx = xs16 - x0.astype(jnp.float32)
                dy = ys16 - y0.astype(jnp.float32)
                x1 = jnp.minimum(x0 + 1, W - 1)
                y1 = jnp.minimum(y0 + 1, H - 1)
                r0 = y0 * W
                r1 = y1 * W
                idxv[0, sl] = r0 + x0
                idxv[1, sl] = r0 + x1
                idxv[2, sl] = r1 + x0
                idxv[3, sl] = r1 + x1
                wv[0, sl] = (1.0 - dx) * (1.0 - dy)
                wv[1, sl] = dx * (1.0 - dy)
                wv[2, sl] = (1.0 - dx) * dy
                wv[3, sl] = dx * dy
            copies = [
                pltpu.async_copy(xt_hbm.at[idxv.at[k]], rows.at[k], sem)
                for k in range(4)
            ]
            for cp in copies:
                cp.wait()

            def grp_body(g, c2):
                pbase = g * L
                w0v = wv[0, pl.ds(pbase, L)]
                w1v = wv[1, pl.ds(pbase, L)]
                w2v = wv[2, pl.ds(pbase, L)]
                w3v = wv[3, pl.ds(pbase, L)]
                for pp in range(L):
                    p = pbase + pp
                    w0 = w0v[pp]
                    w1 = w1v[pp]
                    w2 = w2v[pp]
                    w3 = w3v[pp]
                    for j in range(C // L):
                        slj = pl.ds(j * L, L)
                        outv[p, slj] = (
                            rows[0, p, slj] * w0
                            + rows[1, p, slj] * w1
                            + rows[2, p, slj] * w2
                            + rows[3, p, slj] * w3
                        )
                return c2

            lax.fori_loop(0, P // L, grp_body, 0)
            pltpu.sync_copy(outv, out_hbm.at[pl.ds(base, P)])
            return carry

        lax.fori_loop(0, n_chunks, chunk_body, 0)

    return body


def kernel(x, sample_map):
    B, C, H, W = x.shape
    Ho, Wo = sample_map.shape[0], sample_map.shape[1]
    HW_in = H * W
    HW_out = Ho * Wo
    xt = x.reshape(C, HW_in).T
    xs = sample_map[..., 0].reshape(HW_out)
    ys = sample_map[..., 1].reshape(HW_out)
    sc = _make_sc_resample(HW_in, HW_out, C, H, W)
    out_t = sc(xt, xs, ys)
    return out_t.T.reshape(B, C, Ho, Wo)


# speedup vs baseline: 1.0386x; 1.0386x over previous
import functools

import jax
import jax.numpy as jnp
from jax import lax
from jax.experimental import pallas as pl
from jax.experimental.pallas import tpu as pltpu
from jax.experimental.pallas import tpu_sc as plsc

L = 16
P = 16


def _make_sc_resample(HW_in, HW_out, C, H, W):
    info = plsc.get_sparse_core_info()
    NW = info.num_cores * info.num_subcores
    per_w = HW_out // NW
    n_chunks = per_w // P

    mesh = plsc.VectorSubcoreMesh(core_axis_name="c", subcore_axis_name="s")

    @functools.partial(
        pl.kernel,
        mesh=mesh,
        out_type=jax.ShapeDtypeStruct((HW_out, C), jnp.float32),
        scratch_types=[
            pltpu.VMEM((2, P), jnp.float32),
            pltpu.VMEM((2, P), jnp.float32),
            pltpu.VMEM((2, 4, P), jnp.int32),
            pltpu.VMEM((2, 4, P), jnp.float32),
            pltpu.VMEM((2, 4, P, C), jnp.float32),
            pltpu.VMEM((2, P, C), jnp.float32),
            pltpu.SemaphoreType.DMA,
            pltpu.SemaphoreType.DMA,
            pltpu.SemaphoreType.DMA,
            pltpu.SemaphoreType.DMA,
        ],
    )
    def body(xt_hbm, xs_hbm, ys_hbm, out_hbm,
             xsv, ysv, idxv, wv, rows, outv, g0, g1, o0, o1):
        cid = lax.axis_index("c")
        sid = lax.axis_index("s")
        wid = sid * info.num_cores + cid
        base_w = wid * per_w

        def gsem(slot):
            return [g0, g1][slot]

        def osem(slot):
            return [o0, o1][slot]

        def prep(c, slot):
            base = base_w + c * P
            pltpu.sync_copy(xs_hbm.at[pl.ds(base, P)], xsv.at[slot])
            pltpu.sync_copy(ys_hbm.at[pl.ds(base, P)], ysv.at[slot])
            xs16 = xsv[slot]
            ys16 = ysv[slot]
            x0 = xs16.astype(jnp.int32)
            y0 = ys16.astype(jnp.int32)
            dx = xs16 - x0.astype(jnp.float32)
            dy = ys16 - y0.astype(jnp.float32)
            x1 = jnp.minimum(x0 + 1, W - 1)
            y1 = jnp.minimum(y0 + 1, H - 1)
            r0 = y0 * W
            r1 = y1 * W
            idxv[slot, 0] = r0 + x0
            idxv[slot, 1] = r0 + x1
            idxv[slot, 2] = r1 + x0
            idxv[slot, 3] = r1 + x1
            wv[slot, 0] = (1.0 - dx) * (1.0 - dy)
            wv[slot, 1] = dx * (1.0 - dy)
            wv[slot, 2] = (1.0 - dx) * dy
            wv[slot, 3] = dx * dy

        def fire(slot):
            for k in range(4):
                pltpu.async_copy(
                    xt_hbm.at[idxv.at[slot, k]], rows.at[slot, k], gsem(slot))

        def wait_gathers(slot):
            for k in range(4):
                pltpu.make_async_copy(
                    xt_hbm.at[idxv.at[slot, k]], rows.at[slot, k],
                    gsem(slot)).wait()

        def blend(slot):
            w0v = wv[slot, 0]
            w1v = wv[slot, 1]
            w2v = wv[slot, 2]
            w3v = wv[slot, 3]
            for pp in range(P):
                w0 = w0v[pp]
                w1 = w1v[pp]
                w2 = w2v[pp]
                w3 = w3v[pp]
                for j in range(C // L):
                    slj = pl.ds(j * L, L)
                    outv[slot, pp, slj] = (
                        rows[slot, 0, pp, slj] * w0
                        + rows[slot, 1, pp, slj] * w1
                        + rows[slot, 2, pp, slj] * w2
                        + rows[slot, 3, pp, slj] * w3
                    )

        def issue_out(c, slot):
            base = base_w + c * P
            pltpu.async_copy(outv.at[slot], out_hbm.at[pl.ds(base, P)],
                             osem(slot))

        def drain_out(c, slot):
            base = base_w + c * P
            pltpu.make_async_copy(outv.at[slot],
                                  out_hbm.at[pl.ds(base, P)],
                                  osem(slot)).wait()

        prep(0, 0)
        fire(0)

        def step(i, carry):
            for slot in (0, 1):
                c = 2 * i + slot
                nxt = c + 1

                @pl.when(nxt < n_chunks)
                def _():
                    prep(nxt, 1 - slot)
                    fire(1 - slot)

                wait_gathers(slot)

                @pl.when(c >= 2)
                def _():
                    drain_out(c - 2, slot)

                blend(slot)
                issue_out(c, slot)
            return carry

        lax.fori_loop(0, n_chunks // 2, step, 0)
        drain_out(n_chunks - 2, 0)
        drain_out(n_chunks - 1, 1)

    return body


def kernel(x, sample_map):
    B, C, H, W = x.shape
    Ho, Wo = sample_map.shape[0], sample_map.shape[1]
    HW_in = H * W
    HW_out = Ho * Wo
    xt = x.reshape(C, HW_in).T
    xs = sample_map[..., 0].reshape(HW_out)
    ys = sample_map[..., 1].reshape(HW_out)
    sc = _make_sc_resample(HW_in, HW_out, C, H, W)
    out_t = sc(xt, xs, ys)
    return out_t.T.reshape(B, C, Ho, Wo)


# speedup vs baseline: 2.2196x; 2.1371x over previous
import functools

import jax
import jax.numpy as jnp
from jax import lax
from jax.experimental import pallas as pl
from jax.experimental.pallas import tpu as pltpu
from jax.experimental.pallas import tpu_sc as plsc

L = 16
P = 96


def _make_sc_resample(HW_in, HW_out, C, H, W):
    info = plsc.get_sparse_core_info()
    NW = info.num_cores * info.num_subcores
    per_w = HW_out // NW
    n_chunks = per_w // P

    mesh = plsc.VectorSubcoreMesh(core_axis_name="c", subcore_axis_name="s")

    @functools.partial(
        pl.kernel,
        mesh=mesh,
        out_type=jax.ShapeDtypeStruct((HW_out, C), jnp.float32),
        scratch_types=[
            pltpu.VMEM((per_w,), jnp.float32),
            pltpu.VMEM((per_w,), jnp.float32),
            pltpu.VMEM((2, 4, P), jnp.int32),
            pltpu.VMEM((2, 4, P), jnp.float32),
            pltpu.VMEM((2, P, C), jnp.float32),
            pltpu.VMEM((P, C), jnp.float32),
            pltpu.SemaphoreType.DMA,
            pltpu.SemaphoreType.DMA,
            pltpu.SemaphoreType.DMA,
        ],
    )
    def body(xt_hbm, xs_hbm, ys_hbm, out_hbm,
             xsv, ysv, idxv, wv, rows, outv, g0, g1, osem):
        cid = lax.axis_index("c")
        sid = lax.axis_index("s")
        wid = sid * info.num_cores + cid
        base_w = wid * per_w

        pltpu.sync_copy(xs_hbm.at[pl.ds(base_w, per_w)], xsv)
        pltpu.sync_copy(ys_hbm.at[pl.ds(base_w, per_w)], ysv)

        def gsem(slot):
            return [g0, g1][slot]

        def prep(c, cp):
            for g in range(P // L):
                sl = pl.ds(c * P + g * L, L)
                dsl = pl.ds(g * L, L)
                xs16 = xsv[sl]
                ys16 = ysv[sl]
                x0 = xs16.astype(jnp.int32)
                y0 = ys16.astype(jnp.int32)
                dx = xs16 - x0.astype(jnp.float32)
                dy = ys16 - y0.astype(jnp.float32)
                x1 = jnp.minimum(x0 + 1, W - 1)
                y1 = jnp.minimum(y0 + 1, H - 1)
                r0 = y0 * W
                r1 = y1 * W
                idxv[cp, 0, dsl] = r0 + x0
                idxv[cp, 1, dsl] = r0 + x1
                idxv[cp, 2, dsl] = r1 + x0
                idxv[cp, 3, dsl] = r1 + x1
                wv[cp, 0, dsl] = (1.0 - dx) * (1.0 - dy)
                wv[cp, 1, dsl] = dx * (1.0 - dy)
                wv[cp, 2, dsl] = (1.0 - dx) * dy
                wv[cp, 3, dsl] = dx * dy

        def fire(k, slot, cp):
            pltpu.async_copy(
                xt_hbm.at[idxv.at[cp, k]], rows.at[slot], gsem(slot))

        def wait_gather(k, slot, cp):
            pltpu.make_async_copy(
                xt_hbm.at[idxv.at[cp, k]], rows.at[slot], gsem(slot)).wait()

        def blend(k, slot, cp):
            def grp(g, carry):
                wk16 = wv[cp, k, pl.ds(g * L, L)]
                for pp in range(L):
                    wk = wk16[pp]
                    p = g * L + pp
                    for j in range(C // L):
                        slj = pl.ds(j * L, L)
                        v = rows[slot, p, slj] * wk
                        if k == 0:
                            outv[p, slj] = v
                        else:
                            plsc.addupdate(outv.at[p, slj], v)
                return carry

            lax.fori_loop(0, P // L, grp, 0)

        def issue_out(c):
            pltpu.async_copy(
                outv, out_hbm.at[pl.ds(base_w + c * P, P)], osem)

        def drain_out(c):
            pltpu.make_async_copy(
                outv, out_hbm.at[pl.ds(base_w + c * P, P)], osem).wait()

        prep(0, 0)
        fire(0, 0, 0)

        def chunk_step(c, carry):
            cp = lax.rem(c, 2)
            cpn = 1 - cp
            for k in range(4):
                nslot = k % 2
                if k < 3:
                    fire(k + 1, (k + 1) % 2, cp)
                else:
                    @pl.when(c + 1 < n_chunks)
                    def _():
                        fire(0, 0, cpn)
                if k == 0:
                    @pl.when(c + 1 < n_chunks)
                    def _():
                        prep(c + 1, cpn)
                wait_gather(k, nslot, cp)
                if k == 0:
                    @pl.when(c >= 1)
                    def _():
                        drain_out(c - 1)
                blend(k, nslot, cp)
                if k == 3:
                    issue_out(c)
            return carry

        lax.fori_loop(0, n_chunks, chunk_step, 0)
        drain_out(n_chunks - 1)

    return body


def kernel(x, sample_map):
    B, C, H, W = x.shape
    Ho, Wo = sample_map.shape[0], sample_map.shape[1]
    HW_in = H * W
    HW_out = Ho * Wo
    xt = x.reshape(C, HW_in).T
    xs = sample_map[..., 0].reshape(HW_out)
    ys = sample_map[..., 1].reshape(HW_out)
    sc = _make_sc_resample(HW_in, HW_out, C, H, W)
    out_t = sc(xt, xs, ys)
    return out_t.T.reshape(B, C, Ho, Wo)
